# baseline (device time: 28581 ns/iter reference)
import jax
import jax.numpy as jnp
from jax import lax
from jax.experimental import pallas as pl
from jax.experimental.pallas import tpu as pltpu

N_DEV = 16
W_SLOTS = 8


def kernel(x, w_mat):
    K, kper = x.shape
    _, N = w_mat.shape
    m_per = K // N_DEV
    N2 = N // 2

    def body(x_hbm, w_hbm, out_ref, xv, qsend, qbuf, ssend, sbuf, wbuf, obuf,
             qsend_sems, qrecv_sems, ssend_sems, srecv_sems, wsems, osem,
             xsem, credit_sems):
        my = lax.axis_index("i")
        f32 = jnp.float32

        xdma = pltpu.make_async_copy(x_hbm, xv, xsem)
        xdma.start()
        x_ref = xv

        def w_dmas(u, slot):
            return [
                pltpu.make_async_copy(
                    w_hbm.at[pl.ds(u * kper, kper), pl.ds(h * N2, N2)],
                    wbuf.at[slot, :, pl.ds(h * N2, N2)],
                    wsems.at[slot, h],
                )
                for h in range(2)
            ]

        for s in range(W_SLOTS - 1):
            for d in w_dmas(lax.rem(my - s + N_DEV, N_DEV), s % W_SLOTS):
                d.start()

        for s in range(1, N_DEV):
            d = lax.rem(my - s + N_DEV, N_DEV)
            pl.semaphore_signal(
                credit_sems.at[s], inc=1,
                device_id=(d,), device_id_type=pl.DeviceIdType.MESH,
            )

        barrier_sem = pltpu.get_barrier_semaphore()
        for nbr in (lax.rem(my + 1, N_DEV), lax.rem(my - 1 + N_DEV, N_DEV)):
            pl.semaphore_signal(
                barrier_sem, inc=1,
                device_id=(nbr,), device_id_type=pl.DeviceIdType.MESH,
            )
        pl.semaphore_wait(barrier_sem, 2)

        def quant(s):
            j = lax.rem(my + s, N_DEV)
            chunk = x_ref[pl.ds(j * m_per, m_per), :]
            amax = jnp.maximum(jnp.max(jnp.abs(chunk)), 1e-20)
            q = jnp.round(chunk * (127.0 / amax))
            qsend[s] = q.astype(jnp.int8)
            ssend[s, :] = jnp.full((128,), amax * (1.0 / 127.0), f32)

        sends = []
        xdma.wait()
        quant(1)
        for s in range(1, N_DEV):
            if s + 1 < N_DEV:
                quant(s + 1)
            j = lax.rem(my + s, N_DEV)
            pl.semaphore_wait(credit_sems.at[s], 1)
            for rdma in (
                pltpu.make_async_remote_copy(
                    src_ref=qsend.at[s],
                    dst_ref=qbuf.at[s],
                    send_sem=qsend_sems.at[s],
                    recv_sem=qrecv_sems.at[s],
                    device_id=(j,),
                    device_id_type=pl.DeviceIdType.MESH,
                ),
                pltpu.make_async_remote_copy(
                    src_ref=ssend.at[pl.ds(s, 1), :],
                    dst_ref=sbuf.at[pl.ds(s, 1), :],
                    send_sem=ssend_sems.at[s],
                    recv_sem=srecv_sems.at[s],
                    device_id=(j,),
                    device_id_type=pl.DeviceIdType.MESH,
                ),
            ):
                rdma.start()
                sends.append(rdma)

        acc = jnp.zeros((m_per, N), f32)
        for s in range(N_DEV):
            u = lax.rem(my - s + N_DEV, N_DEV)
            slot = s % W_SLOTS
            if s + W_SLOTS - 1 < N_DEV:
                nxt = s + W_SLOTS - 1
                for d in w_dmas(lax.rem(my - nxt + N_DEV, N_DEV),
                                nxt % W_SLOTS):
                    d.start()
            for d in w_dmas(u, slot):
                d.wait()
            if s == 0:
                xchunk = x_ref[pl.ds(my * m_per, m_per), :]
            else:
                recv_q = pltpu.make_async_remote_copy(
                    src_ref=qsend.at[s],
                    dst_ref=qbuf.at[s],
                    send_sem=qsend_sems.at[s],
                    recv_sem=qrecv_sems.at[s],
                    device_id=(my,),
                    device_id_type=pl.DeviceIdType.MESH,
                )
                recv_q.wait_recv()
                recv_s = pltpu.make_async_remote_copy(
                    src_ref=ssend.at[pl.ds(s, 1), :],
                    dst_ref=sbuf.at[pl.ds(s, 1), :],
                    send_sem=ssend_sems.at[s],
                    recv_sem=srecv_sems.at[s],
                    device_id=(my,),
                    device_id_type=pl.DeviceIdType.MESH,
                )
                recv_s.wait_recv()
                xchunk = qbuf[s].astype(f32) * sbuf[s, 0]
            acc = acc + jnp.dot(
                xchunk, wbuf[slot],
                precision=lax.Precision.DEFAULT,
                preferred_element_type=f32,
            )

        obuf[:, :] = jnp.maximum(acc, 0.0).astype(jnp.bfloat16)
        outdma = pltpu.make_async_copy(obuf, out_ref, osem)
        outdma.start()
        outdma.wait()

        for rdma in sends:
            rdma.wait_send()

    return pl.pallas_call(
        body,
        out_shape=jax.ShapeDtypeStruct((m_per, N), jnp.bfloat16),
        in_specs=[
            pl.BlockSpec(memory_space=pl.ANY),
            pl.BlockSpec(memory_space=pl.ANY),
        ],
        out_specs=pl.BlockSpec(memory_space=pl.ANY),
        scratch_shapes=[
            pltpu.VMEM((K, kper), jnp.float32),
            pltpu.VMEM((N_DEV, m_per, kper), jnp.int8),
            pltpu.VMEM((N_DEV, m_per, kper), jnp.int8),
            pltpu.VMEM((N_DEV, 128), jnp.float32),
            pltpu.VMEM((N_DEV, 128), jnp.float32),
            pltpu.VMEM((W_SLOTS, kper, N), jnp.float32),
            pltpu.VMEM((m_per, N), jnp.bfloat16),
            pltpu.SemaphoreType.DMA((N_DEV,)),
            pltpu.SemaphoreType.DMA((N_DEV,)),
            pltpu.SemaphoreType.DMA((N_DEV,)),
            pltpu.SemaphoreType.DMA((N_DEV,)),
            pltpu.SemaphoreType.DMA((W_SLOTS, 2)),
            pltpu.SemaphoreType.DMA,
            pltpu.SemaphoreType.DMA,
            pltpu.SemaphoreType.REGULAR((N_DEV,)),
        ],
        compiler_params=pltpu.CompilerParams(
            collective_id=0,
            vmem_limit_bytes=60 * 1024 * 1024,
        ),
    )(x, w_mat)


# device time: 27579 ns/iter; 1.0363x vs baseline; 1.0363x over previous
import jax
import jax.numpy as jnp
from jax import lax
from jax.experimental import pallas as pl
from jax.experimental.pallas import tpu as pltpu

N_DEV = 16
W_SLOTS = 6


def kernel(x, w_mat):
    K, kper = x.shape
    _, N = w_mat.shape
    m_per = K // N_DEV
    N2 = N // 2

    def body(x_hbm, w_hbm, out_ref, xv, qsend, qbuf, ssend, sbuf, wbuf, obuf,
             qsend_sems, qrecv_sems, ssend_sems, srecv_sems, wsems, osem,
             xsem, credit_sems):
        my = lax.axis_index("i")
        f32 = jnp.float32

        xdma = pltpu.make_async_copy(x_hbm, xv, xsem)
        xdma.start()
        x_ref = xv

        def w_dmas(u, slot):
            return [
                pltpu.make_async_copy(
                    w_hbm.at[pl.ds(u * kper, kper), pl.ds(h * N2, N2)],
                    wbuf.at[slot, :, pl.ds(h * N2, N2)],
                    wsems.at[slot, h],
                )
                for h in range(2)
            ]

        for s in range(W_SLOTS - 1):
            for d in w_dmas(lax.rem(my - s + N_DEV, N_DEV), s % W_SLOTS):
                d.start()

        for s in range(1, N_DEV):
            d = lax.rem(my - s + N_DEV, N_DEV)
            pl.semaphore_signal(
                credit_sems.at[s], inc=1,
                device_id=(d,), device_id_type=pl.DeviceIdType.MESH,
            )

        barrier_sem = pltpu.get_barrier_semaphore()
        for nbr in (lax.rem(my + 1, N_DEV), lax.rem(my - 1 + N_DEV, N_DEV)):
            pl.semaphore_signal(
                barrier_sem, inc=1,
                device_id=(nbr,), device_id_type=pl.DeviceIdType.MESH,
            )
        pl.semaphore_wait(barrier_sem, 2)

        def quant(s):
            j = lax.rem(my + s, N_DEV)
            chunk = x_ref[pl.ds(j * m_per, m_per), :]
            amax = jnp.maximum(jnp.max(jnp.abs(chunk)), 1e-20)
            q = jnp.round(chunk * (127.0 / amax))
            qsend[s] = q.astype(jnp.int8)
            ssend[s, :] = jnp.full((128,), amax * (1.0 / 127.0), f32)

        sends = []
        xdma.wait()
        quant(1)
        for s in range(1, N_DEV):
            if s + 1 < N_DEV:
                quant(s + 1)
            j = lax.rem(my + s, N_DEV)
            pl.semaphore_wait(credit_sems.at[s], 1)
            for rdma in (
                pltpu.make_async_remote_copy(
                    src_ref=qsend.at[s],
                    dst_ref=qbuf.at[s],
                    send_sem=qsend_sems.at[s],
                    recv_sem=qrecv_sems.at[s],
                    device_id=(j,),
                    device_id_type=pl.DeviceIdType.MESH,
                ),
                pltpu.make_async_remote_copy(
                    src_ref=ssend.at[pl.ds(s, 1), :],
                    dst_ref=sbuf.at[pl.ds(s, 1), :],
                    send_sem=ssend_sems.at[s],
                    recv_sem=srecv_sems.at[s],
                    device_id=(j,),
                    device_id_type=pl.DeviceIdType.MESH,
                ),
            ):
                rdma.start()
                sends.append(rdma)

        acc = jnp.zeros((m_per, N), f32)
        for s in range(N_DEV):
            u = lax.rem(my - s + N_DEV, N_DEV)
            slot = s % W_SLOTS
            if s + W_SLOTS - 1 < N_DEV:
                nxt = s + W_SLOTS - 1
                for d in w_dmas(lax.rem(my - nxt + N_DEV, N_DEV),
                                nxt % W_SLOTS):
                    d.start()
            for d in w_dmas(u, slot):
                d.wait()
            if s == 0:
                xchunk = x_ref[pl.ds(my * m_per, m_per), :]
            else:
                recv_q = pltpu.make_async_remote_copy(
                    src_ref=qsend.at[s],
                    dst_ref=qbuf.at[s],
                    send_sem=qsend_sems.at[s],
                    recv_sem=qrecv_sems.at[s],
                    device_id=(my,),
                    device_id_type=pl.DeviceIdType.MESH,
                )
                recv_q.wait_recv()
                recv_s = pltpu.make_async_remote_copy(
                    src_ref=ssend.at[pl.ds(s, 1), :],
                    dst_ref=sbuf.at[pl.ds(s, 1), :],
                    send_sem=ssend_sems.at[s],
                    recv_sem=srecv_sems.at[s],
                    device_id=(my,),
                    device_id_type=pl.DeviceIdType.MESH,
                )
                recv_s.wait_recv()
                xchunk = qbuf[s].astype(f32) * sbuf[s, 0]
            acc = acc + jnp.dot(
                xchunk, wbuf[slot],
                precision=lax.Precision.DEFAULT,
                preferred_element_type=f32,
            )

        obuf[:, :] = jnp.maximum(acc, 0.0).astype(jnp.bfloat16)
        outdma = pltpu.make_async_copy(obuf, out_ref, osem)
        outdma.start()
        outdma.wait()

        for rdma in sends:
            rdma.wait_send()

    return pl.pallas_call(
        body,
        out_shape=jax.ShapeDtypeStruct((m_per, N), jnp.bfloat16),
        in_specs=[
            pl.BlockSpec(memory_space=pl.ANY),
            pl.BlockSpec(memory_space=pl.ANY),
        ],
        out_specs=pl.BlockSpec(memory_space=pl.ANY),
        scratch_shapes=[
            pltpu.VMEM((K, kper), jnp.float32),
            pltpu.VMEM((N_DEV, m_per, kper), jnp.int8),
            pltpu.VMEM((N_DEV, m_per, kper), jnp.int8),
            pltpu.VMEM((N_DEV, 128), jnp.float32),
            pltpu.VMEM((N_DEV, 128), jnp.float32),
            pltpu.VMEM((W_SLOTS, kper, N), jnp.float32),
            pltpu.VMEM((m_per, N), jnp.bfloat16),
            pltpu.SemaphoreType.DMA((N_DEV,)),
            pltpu.SemaphoreType.DMA((N_DEV,)),
            pltpu.SemaphoreType.DMA((N_DEV,)),
            pltpu.SemaphoreType.DMA((N_DEV,)),
            pltpu.SemaphoreType.DMA((W_SLOTS, 2)),
            pltpu.SemaphoreType.DMA,
            pltpu.SemaphoreType.DMA,
            pltpu.SemaphoreType.REGULAR((N_DEV,)),
        ],
        compiler_params=pltpu.CompilerParams(
            collective_id=0,
            vmem_limit_bytes=60 * 1024 * 1024,
        ),
    )(x, w_mat)
